# baseline (speedup 1.0000x reference)
import functools
import math

import jax
import jax.numpy as jnp
from jax import lax
from jax.experimental import pallas as pl
from jax.experimental.pallas import tpu as pltpu
from jax.experimental.pallas import tpu_sc as plsc

DMODEL = 64
SCALE = math.sqrt(DMODEL)

_NC = 2
_NS = 16
_NW = _NC * _NS

_CHUNK = 128


def _make_lookup(batch: int):
    b_per_w = batch // _NW
    n_chunks = b_per_w // _CHUNK
    mesh = plsc.VectorSubcoreMesh(core_axis_name="c", subcore_axis_name="s")

    @functools.partial(
        pl.kernel,
        out_type=jax.ShapeDtypeStruct((batch, DMODEL), jnp.float32),
        mesh=mesh,
        scratch_types=[
            pltpu.VMEM((b_per_w,), jnp.int32),
            pltpu.VMEM((_CHUNK, DMODEL), jnp.float32),
            pltpu.SemaphoreType.DMA,
        ],
        compiler_params=pltpu.CompilerParams(use_tc_tiling_on_sc=False),
    )
    def lookup(idx_hbm, table_hbm, out_hbm, idx_v, rows_v, sem):
        wid = lax.axis_index("s") * _NC + lax.axis_index("c")
        base = wid * b_per_w
        pltpu.sync_copy(idx_hbm.at[pl.ds(base, b_per_w)], idx_v)

        def chunk_body(c, carry):
            pltpu.async_copy(
                table_hbm.at[idx_v.at[pl.ds(c * _CHUNK, _CHUNK)]],
                rows_v, sem).wait()

            def scale_row(i, carry2):
                for j in range(DMODEL // 16):
                    sl = pl.ds(j * 16, 16)
                    rows_v[i, sl] = rows_v[i, sl] * SCALE
                return carry2

            lax.fori_loop(0, _CHUNK, scale_row, 0, unroll=False)

            pltpu.sync_copy(rows_v,
                            out_hbm.at[pl.ds(base + c * _CHUNK, _CHUNK)])
            return carry

        lax.fori_loop(0, n_chunks, chunk_body, 0, unroll=False)

    return lookup


def kernel(x, table):
    b0, b1 = x.shape
    idx = x.reshape(-1).astype(jnp.int32)
    out = _make_lookup(idx.shape[0])(idx, table)
    return out.reshape(b0, b1, DMODEL)


# speedup vs baseline: 1.2052x; 1.2052x over previous
import functools
import math

import jax
import jax.numpy as jnp
from jax import lax
from jax.experimental import pallas as pl
from jax.experimental.pallas import tpu as pltpu
from jax.experimental.pallas import tpu_sc as plsc

DMODEL = 64
SCALE = math.sqrt(DMODEL)

_NC = 2
_NS = 16
_NW = _NC * _NS

_CHUNK = 128
_NBUF = 4


def _make_lookup(batch: int):
    b_per_w = batch // _NW
    n_chunks = b_per_w // _CHUNK
    n_iter = n_chunks // _NBUF
    mesh = plsc.VectorSubcoreMesh(core_axis_name="c", subcore_axis_name="s")

    @functools.partial(
        pl.kernel,
        out_type=jax.ShapeDtypeStruct((batch, DMODEL), jnp.float32),
        mesh=mesh,
        scratch_types=(
            [pltpu.VMEM((b_per_w,), jnp.int32)]
            + [pltpu.VMEM((_CHUNK, DMODEL), jnp.float32)] * _NBUF
            + [pltpu.SemaphoreType.DMA] * (2 * _NBUF)
        ),
        compiler_params=pltpu.CompilerParams(use_tc_tiling_on_sc=False),
    )
    def lookup(idx_hbm, table_hbm, out_hbm, idx_v, *bufs_and_sems):
        rows = bufs_and_sems[:_NBUF]
        gsem = bufs_and_sems[_NBUF:2 * _NBUF]
        ssem = bufs_and_sems[2 * _NBUF:]
        wid = lax.axis_index("s") * _NC + lax.axis_index("c")
        base = wid * b_per_w
        pltpu.sync_copy(idx_hbm.at[pl.ds(base, b_per_w)], idx_v)

        def start_gather(c, b):
            pltpu.async_copy(
                table_hbm.at[idx_v.at[pl.ds(c * _CHUNK, _CHUNK)]],
                rows[b], gsem[b])

        def wait_gather(b):
            pltpu.make_async_copy(
                table_hbm.at[idx_v.at[pl.ds(0, _CHUNK)]],
                rows[b], gsem[b]).wait()

        def start_store(c, b):
            pltpu.async_copy(rows[b],
                             out_hbm.at[pl.ds(base + c * _CHUNK, _CHUNK)],
                             ssem[b])

        def wait_store(b):
            pltpu.make_async_copy(rows[b],
                                  out_hbm.at[pl.ds(base, _CHUNK)],
                                  ssem[b]).wait()

        def scale_buf(b):
            def scale_row(i, carry):
                for j in range(DMODEL // 16):
                    sl = pl.ds(j * 16, 16)
                    rows[b][i, sl] = rows[b][i, sl] * SCALE
                return carry

            lax.fori_loop(0, _CHUNK, scale_row, 0, unroll=8)

        for b in range(_NBUF):
            start_gather(b, b)

        def body(i, carry):
            for b in range(_NBUF):
                c = i * _NBUF + b
                wait_gather(b)
                scale_buf(b)
                start_store(c, b)
                wait_store(b)
                start_gather(c + _NBUF, b)
            return carry

        lax.fori_loop(0, n_iter - 1, body, 0, unroll=False)

        for b in range(_NBUF):
            wait_gather(b)
            scale_buf(b)
            start_store((n_iter - 1) * _NBUF + b, b)
        for b in range(_NBUF):
            wait_store(b)

    return lookup


def kernel(x, table):
    b0, b1 = x.shape
    idx = x.reshape(-1).astype(jnp.int32)
    out = _make_lookup(idx.shape[0])(idx, table)
    return out.reshape(b0, b1, DMODEL)
